# baseline (speedup 1.0000x reference)
import functools

import jax
import jax.numpy as jnp
from jax import lax
from jax.experimental import pallas as pl
from jax.experimental.pallas import tpu as pltpu
from jax.experimental.pallas import tpu_sc as plsc

_CHUNK = 128


@functools.cache
def _build(B, Vp, Dp):
    info = plsc.get_sparse_core_info()
    nc, ns = info.num_cores, info.num_subcores
    nw = nc * ns
    b_per_w = B // nw
    n_chunks = b_per_w // _CHUNK

    mesh = plsc.VectorSubcoreMesh(core_axis_name="c", subcore_axis_name="s")

    @functools.partial(
        pl.kernel,
        mesh=mesh,
        out_type=jax.ShapeDtypeStruct((nw, n_chunks, _CHUNK, Dp), jnp.float32),
        scratch_types=[
            pltpu.VMEM((n_chunks, _CHUNK), jnp.int32),
            pltpu.VMEM((n_chunks, _CHUNK, Dp), jnp.float32),
            pltpu.SemaphoreType.DMA,
        ],
    )
    def gather_kernel(idx_hbm, table_hbm, out_hbm, idx_v, rows_v, sem):
        wid = lax.axis_index("s") * nc + lax.axis_index("c")
        pltpu.sync_copy(idx_hbm.at[wid], idx_v)
        copies = [
            pltpu.async_copy(table_hbm.at[idx_v.at[j]], rows_v.at[j], sem)
            for j in range(n_chunks)
        ]
        for c in copies:
            c.wait()
        pltpu.sync_copy(rows_v, out_hbm.at[wid])

    return gather_kernel, nw, n_chunks


def kernel(y, genre_emb):
    (B,) = y.shape
    V, D = genre_emb.shape
    table2 = genre_emb.reshape(V // 2, 2 * D)
    gather_kernel, nw, n_chunks = _build(B, V // 2, 2 * D)
    idx = y.astype(jnp.int32)
    pair = (idx >> 1).reshape(nw, n_chunks, _CHUNK)
    out2 = gather_kernel(pair, table2).reshape(B, 2, D)
    par = (idx & 1)[:, None, None]
    out = jnp.where(par == 0, out2[:, 0:1, :], out2[:, 1:2, :])
    return out


# speedup vs baseline: 1.7606x; 1.7606x over previous
import functools

import jax
import jax.numpy as jnp
from jax import lax
from jax.experimental import pallas as pl
from jax.experimental.pallas import tpu as pltpu
from jax.experimental.pallas import tpu_sc as plsc


@functools.cache
def _build(B, V, D):
    info = plsc.get_sparse_core_info()
    nc, ns, nl = info.num_cores, info.num_subcores, info.num_lanes
    nw = nc * ns
    b_per_w = B // nw

    mesh = plsc.VectorSubcoreMesh(core_axis_name="c", subcore_axis_name="s")

    @functools.partial(
        pl.kernel,
        mesh=mesh,
        out_type=jax.ShapeDtypeStruct((nw, b_per_w, D), jnp.float32),
        scratch_types=[
            pltpu.VMEM((b_per_w,), jnp.int32),
            pltpu.VMEM((b_per_w, D), jnp.float32),
            pltpu.SemaphoreType.DMA,
        ],
    )
    def gather_kernel(idx_hbm, table_hbm, out_hbm, idx_v, rows_v, sem):
        wid = lax.axis_index("s") * nc + lax.axis_index("c")
        pltpu.sync_copy(idx_hbm.at[wid], idx_v)

        @pl.loop(0, b_per_w // nl)
        def _grp(g):
            vals = idx_v[pl.ds(g * nl, nl)]
            for l in range(nl):
                r = vals[l]
                pltpu.make_async_copy(
                    table_hbm.at[pl.ds(r, 1), :],
                    rows_v.at[pl.ds(g * nl + l, 1)],
                    sem,
                ).start()

        pltpu.make_async_copy(
            table_hbm.at[pl.ds(0, b_per_w), :], rows_v, sem
        ).wait()
        pltpu.sync_copy(rows_v, out_hbm.at[wid])

    return gather_kernel, nw


def kernel(y, genre_emb):
    (B,) = y.shape
    V, D = genre_emb.shape
    gather_kernel, nw = _build(B, V, D)
    idx = y.astype(jnp.int32).reshape(nw, B // nw)
    out = gather_kernel(idx, genre_emb)
    return out.reshape(B, 1, D)
